# baseline (speedup 1.0000x reference)
import functools

import jax
import jax.numpy as jnp
from jax import lax
from jax.experimental import pallas as pl
from jax.experimental.pallas import tpu as pltpu
from jax.experimental.pallas import tpu_sc as plsc

B = 2
H = 64
W = 64
NP = B * H * W
NW = 32
PW = NP // NW
NG = PW // 16
NKR = 40
NIDX = NKR * 48
NCH = 240
NCHUNK = NIDX // NCH
NROWS = (NP * H * W) // 8
OUT_F = 81

_f32 = jnp.float32
_i32 = jnp.int32


def _sc_body(cm2, x0h, y0h, x1h, y1h, out,
             x0v, y0v, x1v, y1v, idx_v, rows_v, p_v, t_v, sem0, sem1):
    wid = lax.axis_index("s") * 2 + lax.axis_index("c")
    base = wid * PW

    pltpu.sync_copy(x0h.at[pl.ds(base, PW)], x0v)
    pltpu.sync_copy(y0h.at[pl.ds(base, PW)], y0v)
    pltpu.sync_copy(x1h.at[pl.ds(base, PW)], x1v)
    pltpu.sync_copy(y1h.at[pl.ds(base, PW)], y1v)

    lane = lax.iota(_i32, 16)

    def build_and_fire(g, po, sem_even, sem_odd):
        off = g * 16
        gx0 = x0v[pl.ds(off, 16)]
        gy0 = y0v[pl.ds(off, 16)]
        gx1 = x1v[pl.ds(off, 16)]
        gy1 = y1v[pl.ds(off, 16)]

        pvec = base + off + lane
        b_addr = (pvec >> 12) << 12

        ix1 = gx1.astype(_i32)
        iy1 = gy1.astype(_i32)
        a0 = (iy1 * 64 + ix1) * 8192 + b_addr
        ox = gx0.astype(_i32) - 4
        oy = gy0.astype(_i32) - 4
        wlo = jnp.clip(ox, 0, 54)

        hw = []
        for r in range(10):
            hw.append(jnp.clip(oy + r, 0, 63) * 64 + wlo)

        acorn = [a0, a0 + 524288, a0 + 8192, a0 + 532480]
        for k in range(4):
            for r in range(10):
                kr = k * 10 + r
                r0 = (acorn[k] + hw[r]) >> 3
                idx_v[pl.ds(po + kr * 48, 16)] = r0
                idx_v[pl.ds(po + kr * 48 + 16, 16)] = jnp.minimum(r0 + 1, NROWS - 1)
                idx_v[pl.ds(po + kr * 48 + 32, 16)] = jnp.minimum(r0 + 2, NROWS - 1)

        even = (g & 1) == 0

        @pl.when(even)
        def _():
            def fire(i, c2):
                pltpu.async_copy(cm2.at[idx_v.at[pl.ds(po + i * NCH, NCH)]],
                                 rows_v.at[pl.ds(po + i * NCH, NCH)], sem_even)
                return c2
            lax.fori_loop(0, NCHUNK, fire, 0)

        @pl.when(jnp.logical_not(even))
        def _():
            def fire(i, c2):
                pltpu.async_copy(cm2.at[idx_v.at[pl.ds(po + i * NCH, NCH)]],
                                 rows_v.at[pl.ds(po + i * NCH, NCH)], sem_odd)
                return c2
            lax.fori_loop(0, NCHUNK, fire, 0)

    def compute(g, po):
        off = g * 16
        gx0 = x0v[pl.ds(off, 16)]
        gy0 = y0v[pl.ds(off, 16)]
        gx1 = x1v[pl.ds(off, 16)]
        gy1 = y1v[pl.ds(off, 16)]

        ix1 = gx1.astype(_i32)
        iy1 = gy1.astype(_i32)
        fx1 = gx1 - ix1.astype(_f32)
        fy1 = gy1 - iy1.astype(_f32)
        wa = (1.0 - fx1) * (1.0 - fy1)
        wb = (1.0 - fx1) * fy1
        wc = fx1 * (1.0 - fy1)
        wd = fx1 * fy1

        ix0 = gx0.astype(_i32)
        iy0 = gy0.astype(_i32)
        fx0 = gx0 - ix0.astype(_f32)
        fy0 = gy0 - iy0.astype(_f32)
        ox = ix0 - 4
        oy = iy0 - 4
        q00 = (1.0 - fx0) * (1.0 - fy0)
        q01 = (1.0 - fx0) * fy0
        q10 = fx0 * (1.0 - fy0)
        q11 = fx0 * fy0

        wlo = jnp.clip(ox, 0, 54)
        sf = wlo & 7

        mcol = []
        rowadd = []
        cols = []
        for c in range(10):
            w1 = ox + c
            mcol.append(jnp.where((w1 >= 0) & (w1 <= 63), 1.0, 0.0).astype(_f32))
            offc = sf + (jnp.clip(w1, 0, 63) - wlo)
            rowadd.append(po + lane + ((offc >> 3) << 4))
            cols.append(offc & 7)

        mrow = []
        for r in range(10):
            h1 = oy + r
            mrow.append(jnp.where((h1 >= 0) & (h1 <= 63), 1.0, 0.0).astype(_f32))

        for r in range(10):
            for c in range(10):
                g0 = plsc.load_gather(rows_v, [rowadd[c] + (0 + r * 48), cols[c]])
                g1 = plsc.load_gather(rows_v, [rowadd[c] + (480 + r * 48), cols[c]])
                g2 = plsc.load_gather(rows_v, [rowadd[c] + (960 + r * 48), cols[c]])
                g3 = plsc.load_gather(rows_v, [rowadd[c] + (1440 + r * 48), cols[c]])
                acc = ((wa * g0 + wb * g1) + (wc * g2 + wd * g3)) * (mrow[r] * mcol[c])
                p_v[pl.ds((r * 10 + c) * 16, 16)] = acc

        for a in range(9):
            for b2 in range(9):
                p00 = p_v[pl.ds((b2 * 10 + a) * 16, 16)]
                p01 = p_v[pl.ds(((b2 + 1) * 10 + a) * 16, 16)]
                p10 = p_v[pl.ds((b2 * 10 + a + 1) * 16, 16)]
                p11 = p_v[pl.ds(((b2 + 1) * 10 + a + 1) * 16, 16)]
                o = (q00 * p00 + q01 * p01) + (q10 * p10 + q11 * p11)
                t_v[a * 9 + b2, pl.ds(off, 16)] = o

    def step(g, carry):
        po_g = (g & 1) * NIDX

        @pl.when(g < NG)
        def _():
            build_and_fire(g, po_g, sem0, sem1)

        gc = g - 1
        po_c = (gc & 1) * NIDX

        @pl.when((g >= 1) & ((g & 1) == 1))
        def _():
            pltpu.make_async_copy(cm2.at[pl.ds(0, NIDX)],
                                  rows_v.at[pl.ds(0, NIDX)], sem0).wait()

        @pl.when((g >= 1) & ((g & 1) == 0))
        def _():
            pltpu.make_async_copy(cm2.at[pl.ds(0, NIDX)],
                                  rows_v.at[pl.ds(NIDX, NIDX)], sem1).wait()

        @pl.when(g >= 1)
        def _():
            compute(gc, po_c)

        return carry

    lax.fori_loop(0, NG + 1, step, 0)
    pltpu.sync_copy(t_v, out.at[:, pl.ds(base, PW)])


@jax.jit
def _run(cm2, x0, y0, x1, y1):
    mesh = plsc.VectorSubcoreMesh(core_axis_name="c", subcore_axis_name="s",
                                  num_cores=2, num_subcores=16)
    f = pl.kernel(
        _sc_body,
        out_type=jax.ShapeDtypeStruct((OUT_F, NP), _f32),
        mesh=mesh,
        compiler_params=pltpu.CompilerParams(use_tc_tiling_on_sc=False,
                                             needs_layout_passes=False),
        scratch_types=[
            pltpu.VMEM((PW,), _f32),
            pltpu.VMEM((PW,), _f32),
            pltpu.VMEM((PW,), _f32),
            pltpu.VMEM((PW,), _f32),
            pltpu.VMEM((2 * NIDX,), _i32),
            pltpu.VMEM((2 * NIDX, 8), _f32),
            pltpu.VMEM((1600,), _f32),
            pltpu.VMEM((OUT_F, PW), _f32),
            pltpu.SemaphoreType.DMA,
            pltpu.SemaphoreType.DMA,
        ],
    )
    return f(cm2, x0, y0, x1, y1)


def kernel(cost_maps, coords0, coords1):
    cm2 = jnp.transpose(cost_maps.reshape(NP, H, W), (1, 2, 0)).reshape(NROWS, 8)
    x0 = coords0[:, 0].reshape(-1)
    y0 = coords0[:, 1].reshape(-1)
    x1 = coords1[:, 0].reshape(-1)
    y1 = coords1[:, 1].reshape(-1)
    out = _run(cm2, x0, y0, x1, y1)
    return out.reshape(OUT_F, B, H * W).transpose(1, 0, 2).reshape(B, OUT_F, H, W)


# speedup vs baseline: 1.0183x; 1.0183x over previous
import functools

import jax
import jax.numpy as jnp
from jax import lax
from jax.experimental import pallas as pl
from jax.experimental.pallas import tpu as pltpu
from jax.experimental.pallas import tpu_sc as plsc

B = 2
H = 64
W = 64
NP = B * H * W
NW = 32
PW = NP // NW
NG = PW // 16
NKR = 40
NIDX = NKR * 32
NCH = 256
NCHUNK = NIDX // NCH
NROWS = (NP * H * W) // 16
OUT_F = 81

_f32 = jnp.float32
_i32 = jnp.int32


def _sc_body(cm2, x0h, y0h, x1h, y1h, out,
             x0v, y0v, x1v, y1v, idx_v, rows_v, p_v, t_v, sem0, sem1):
    wid = lax.axis_index("s") * 2 + lax.axis_index("c")
    base = wid * PW

    pltpu.sync_copy(x0h.at[pl.ds(base, PW)], x0v)
    pltpu.sync_copy(y0h.at[pl.ds(base, PW)], y0v)
    pltpu.sync_copy(x1h.at[pl.ds(base, PW)], x1v)
    pltpu.sync_copy(y1h.at[pl.ds(base, PW)], y1v)

    lane = lax.iota(_i32, 16)

    def build_and_fire(g, po, sem_even, sem_odd):
        off = g * 16
        gx0 = x0v[pl.ds(off, 16)]
        gy0 = y0v[pl.ds(off, 16)]
        gx1 = x1v[pl.ds(off, 16)]
        gy1 = y1v[pl.ds(off, 16)]

        pvec = base + off + lane
        b_addr = (pvec >> 12) << 12

        ix1 = gx1.astype(_i32)
        iy1 = gy1.astype(_i32)
        a0 = (iy1 * 64 + ix1) * 8192 + b_addr
        ox = gx0.astype(_i32) - 4
        oy = gy0.astype(_i32) - 4
        wlo = jnp.clip(ox, 0, 54)

        hw = []
        for r in range(10):
            hw.append(jnp.clip(oy + r, 0, 63) * 64 + wlo)

        acorn = [a0, a0 + 524288, a0 + 8192, a0 + 532480]
        for k in range(4):
            for r in range(10):
                kr = k * 10 + r
                r0 = (acorn[k] + hw[r]) >> 4
                idx_v[pl.ds(po + kr * 32, 16)] = r0
                idx_v[pl.ds(po + kr * 32 + 16, 16)] = jnp.minimum(r0 + 1, NROWS - 1)

        even = (g & 1) == 0

        @pl.when(even)
        def _():
            def fire(i, c2):
                pltpu.async_copy(cm2.at[idx_v.at[pl.ds(po + i * NCH, NCH)]],
                                 rows_v.at[pl.ds(po + i * NCH, NCH)], sem_even)
                return c2
            lax.fori_loop(0, NCHUNK, fire, 0)

        @pl.when(jnp.logical_not(even))
        def _():
            def fire(i, c2):
                pltpu.async_copy(cm2.at[idx_v.at[pl.ds(po + i * NCH, NCH)]],
                                 rows_v.at[pl.ds(po + i * NCH, NCH)], sem_odd)
                return c2
            lax.fori_loop(0, NCHUNK, fire, 0)

    def compute(g, po):
        off = g * 16
        gx0 = x0v[pl.ds(off, 16)]
        gy0 = y0v[pl.ds(off, 16)]
        gx1 = x1v[pl.ds(off, 16)]
        gy1 = y1v[pl.ds(off, 16)]

        ix1 = gx1.astype(_i32)
        iy1 = gy1.astype(_i32)
        fx1 = gx1 - ix1.astype(_f32)
        fy1 = gy1 - iy1.astype(_f32)
        wa = (1.0 - fx1) * (1.0 - fy1)
        wb = (1.0 - fx1) * fy1
        wc = fx1 * (1.0 - fy1)
        wd = fx1 * fy1

        ix0 = gx0.astype(_i32)
        iy0 = gy0.astype(_i32)
        fx0 = gx0 - ix0.astype(_f32)
        fy0 = gy0 - iy0.astype(_f32)
        ox = ix0 - 4
        oy = iy0 - 4
        q00 = (1.0 - fx0) * (1.0 - fy0)
        q01 = (1.0 - fx0) * fy0
        q10 = fx0 * (1.0 - fy0)
        q11 = fx0 * fy0

        wlo = jnp.clip(ox, 0, 54)
        sf = wlo & 15

        mcol = []
        rowadd = []
        cols = []
        for c in range(10):
            w1 = ox + c
            mcol.append(jnp.where((w1 >= 0) & (w1 <= 63), 1.0, 0.0).astype(_f32))
            offc = sf + (jnp.clip(w1, 0, 63) - wlo)
            rowadd.append(po + lane + ((offc >> 4) << 4))
            cols.append(offc & 15)

        mrow = []
        for r in range(10):
            h1 = oy + r
            mrow.append(jnp.where((h1 >= 0) & (h1 <= 63), 1.0, 0.0).astype(_f32))

        for r in range(10):
            for c in range(10):
                g0 = plsc.load_gather(rows_v, [rowadd[c] + (0 + r * 32), cols[c]])
                g1 = plsc.load_gather(rows_v, [rowadd[c] + (320 + r * 32), cols[c]])
                g2 = plsc.load_gather(rows_v, [rowadd[c] + (640 + r * 32), cols[c]])
                g3 = plsc.load_gather(rows_v, [rowadd[c] + (960 + r * 32), cols[c]])
                acc = ((wa * g0 + wb * g1) + (wc * g2 + wd * g3)) * (mrow[r] * mcol[c])
                p_v[pl.ds((r * 10 + c) * 16, 16)] = acc

        for a in range(9):
            for b2 in range(9):
                p00 = p_v[pl.ds((b2 * 10 + a) * 16, 16)]
                p01 = p_v[pl.ds(((b2 + 1) * 10 + a) * 16, 16)]
                p10 = p_v[pl.ds((b2 * 10 + a + 1) * 16, 16)]
                p11 = p_v[pl.ds(((b2 + 1) * 10 + a + 1) * 16, 16)]
                o = (q00 * p00 + q01 * p01) + (q10 * p10 + q11 * p11)
                t_v[a * 9 + b2, pl.ds(off, 16)] = o

    def step(g, carry):
        po_g = (g & 1) * NIDX

        @pl.when(g < NG)
        def _():
            build_and_fire(g, po_g, sem0, sem1)

        gc = g - 1
        po_c = (gc & 1) * NIDX

        @pl.when((g >= 1) & ((g & 1) == 1))
        def _():
            pltpu.make_async_copy(cm2.at[pl.ds(0, NIDX)],
                                  rows_v.at[pl.ds(0, NIDX)], sem0).wait()

        @pl.when((g >= 1) & ((g & 1) == 0))
        def _():
            pltpu.make_async_copy(cm2.at[pl.ds(0, NIDX)],
                                  rows_v.at[pl.ds(NIDX, NIDX)], sem1).wait()

        @pl.when(g >= 1)
        def _():
            compute(gc, po_c)

        return carry

    lax.fori_loop(0, NG + 1, step, 0)
    pltpu.sync_copy(t_v, out.at[:, pl.ds(base, PW)])


@jax.jit
def _run(cm2, x0, y0, x1, y1):
    mesh = plsc.VectorSubcoreMesh(core_axis_name="c", subcore_axis_name="s",
                                  num_cores=2, num_subcores=16)
    f = pl.kernel(
        _sc_body,
        out_type=jax.ShapeDtypeStruct((OUT_F, NP), _f32),
        mesh=mesh,
        compiler_params=pltpu.CompilerParams(use_tc_tiling_on_sc=False,
                                             needs_layout_passes=False),
        scratch_types=[
            pltpu.VMEM((PW,), _f32),
            pltpu.VMEM((PW,), _f32),
            pltpu.VMEM((PW,), _f32),
            pltpu.VMEM((PW,), _f32),
            pltpu.VMEM((2 * NIDX,), _i32),
            pltpu.VMEM((2 * NIDX, 16), _f32),
            pltpu.VMEM((1600,), _f32),
            pltpu.VMEM((OUT_F, PW), _f32),
            pltpu.SemaphoreType.DMA,
            pltpu.SemaphoreType.DMA,
        ],
    )
    return f(cm2, x0, y0, x1, y1)


def kernel(cost_maps, coords0, coords1):
    cm2 = jnp.transpose(cost_maps.reshape(NP, H, W), (1, 2, 0)).reshape(NROWS, 16)
    x0 = coords0[:, 0].reshape(-1)
    y0 = coords0[:, 1].reshape(-1)
    x1 = coords1[:, 0].reshape(-1)
    y1 = coords1[:, 1].reshape(-1)
    out = _run(cm2, x0, y0, x1, y1)
    return out.reshape(OUT_F, B, H * W).transpose(1, 0, 2).reshape(B, OUT_F, H, W)


# speedup vs baseline: 1.0212x; 1.0028x over previous
import jax
import jax.numpy as jnp
from jax import lax
from jax.experimental import pallas as pl
from jax.experimental.pallas import tpu as pltpu
from jax.experimental.pallas import tpu_sc as plsc

B = 2
H = 64
W = 64
NP = B * H * W
NW = 32
PW = NP // NW
NG = PW // 16
NKR = 40
NIDX = NKR * 32
NCH = 256
NCHUNK = NIDX // NCH
NROWS = (NP * H * W) // 16
OUT_F = 81

_f32 = jnp.float32
_i32 = jnp.int32


def _sc_body(cm2, x0h, y0h, x1h, y1h, out,
             x0v, y0v, x1v, y1v, idx_v, rows_v, p_v, t_v, sem0, sem1):
    wid = lax.axis_index("s") * 2 + lax.axis_index("c")
    base = wid * PW

    pltpu.sync_copy(x0h.at[pl.ds(base, PW)], x0v)
    pltpu.sync_copy(y0h.at[pl.ds(base, PW)], y0v)
    pltpu.sync_copy(x1h.at[pl.ds(base, PW)], x1v)
    pltpu.sync_copy(y1h.at[pl.ds(base, PW)], y1v)

    lane = lax.iota(_i32, 16)

    def build_and_fire(g, po, sem_even, sem_odd):
        off = g * 16
        gx0 = x0v[pl.ds(off, 16)]
        gy0 = y0v[pl.ds(off, 16)]
        gx1 = x1v[pl.ds(off, 16)]
        gy1 = y1v[pl.ds(off, 16)]

        pvec = base + off + lane
        b_addr = (pvec >> 12) << 12

        ix1 = gx1.astype(_i32)
        iy1 = gy1.astype(_i32)
        a0 = (iy1 * 64 + ix1) * 8192 + b_addr
        ox = gx0.astype(_i32) - 4
        oy = gy0.astype(_i32) - 4
        wlo = jnp.clip(ox, 0, 54)

        hw = []
        for r in range(10):
            hw.append(jnp.clip(oy + r, 0, 63) * 64 + wlo)

        acorn = [a0, a0 + 524288, a0 + 8192, a0 + 532480]
        for k in range(4):
            for r in range(10):
                kr = k * 10 + r
                r0 = (acorn[k] + hw[r]) >> 4
                idx_v[pl.ds(po + kr * 32, 16)] = r0
                idx_v[pl.ds(po + kr * 32 + 16, 16)] = jnp.minimum(r0 + 1, NROWS - 1)

        even = (g & 1) == 0

        @pl.when(even)
        def _():
            def fire(i, c2):
                pltpu.async_copy(cm2.at[idx_v.at[pl.ds(po + i * NCH, NCH)]],
                                 rows_v.at[pl.ds(po + i * NCH, NCH)], sem_even)
                return c2
            lax.fori_loop(0, NCHUNK, fire, 0)

        @pl.when(jnp.logical_not(even))
        def _():
            def fire(i, c2):
                pltpu.async_copy(cm2.at[idx_v.at[pl.ds(po + i * NCH, NCH)]],
                                 rows_v.at[pl.ds(po + i * NCH, NCH)], sem_odd)
                return c2
            lax.fori_loop(0, NCHUNK, fire, 0)

    def compute(g, po):
        off = g * 16
        gx0 = x0v[pl.ds(off, 16)]
        gy0 = y0v[pl.ds(off, 16)]
        gx1 = x1v[pl.ds(off, 16)]
        gy1 = y1v[pl.ds(off, 16)]

        ix1 = gx1.astype(_i32)
        iy1 = gy1.astype(_i32)
        fx1 = gx1 - ix1.astype(_f32)
        fy1 = gy1 - iy1.astype(_f32)
        wa = (1.0 - fx1) * (1.0 - fy1)
        wb = (1.0 - fx1) * fy1
        wc = fx1 * (1.0 - fy1)
        wd = fx1 * fy1

        ix0 = gx0.astype(_i32)
        iy0 = gy0.astype(_i32)
        fx0 = gx0 - ix0.astype(_f32)
        fy0 = gy0 - iy0.astype(_f32)
        ox = ix0 - 4
        oy = iy0 - 4
        q00 = (1.0 - fx0) * (1.0 - fy0)
        q01 = (1.0 - fx0) * fy0
        q10 = fx0 * (1.0 - fy0)
        q11 = fx0 * fy0

        wlo = jnp.clip(ox, 0, 54)
        sf = wlo & 15

        mcol = []
        rowadd = []
        cols = []
        for c in range(10):
            w1 = ox + c
            mcol.append(jnp.where((w1 >= 0) & (w1 <= 63), 1.0, 0.0).astype(_f32))
            offc = sf + (jnp.clip(w1, 0, 63) - wlo)
            rowadd.append(po + lane + ((offc >> 4) << 4))
            cols.append(offc & 15)

        mrow = []
        for r in range(10):
            h1 = oy + r
            mrow.append(jnp.where((h1 >= 0) & (h1 <= 63), 1.0, 0.0).astype(_f32))

        for r in range(10):
            for c in range(10):
                g0 = plsc.load_gather(rows_v, [rowadd[c] + (0 + r * 32), cols[c]])
                g1 = plsc.load_gather(rows_v, [rowadd[c] + (320 + r * 32), cols[c]])
                g2 = plsc.load_gather(rows_v, [rowadd[c] + (640 + r * 32), cols[c]])
                g3 = plsc.load_gather(rows_v, [rowadd[c] + (960 + r * 32), cols[c]])
                acc = ((wa * g0 + wb * g1) + (wc * g2 + wd * g3)) * (mrow[r] * mcol[c])
                p_v[pl.ds((r * 10 + c) * 16, 16)] = acc

        for a in range(9):
            for b2 in range(9):
                p00 = p_v[pl.ds((b2 * 10 + a) * 16, 16)]
                p01 = p_v[pl.ds(((b2 + 1) * 10 + a) * 16, 16)]
                p10 = p_v[pl.ds((b2 * 10 + a + 1) * 16, 16)]
                p11 = p_v[pl.ds(((b2 + 1) * 10 + a + 1) * 16, 16)]
                o = (q00 * p00 + q01 * p01) + (q10 * p10 + q11 * p11)
                t_v[a * 9 + b2, pl.ds(off, 16)] = o

    def step(g, carry):
        po_g = (g & 1) * NIDX

        @pl.when(g < NG)
        def _():
            build_and_fire(g, po_g, sem0, sem1)

        gc = g - 1
        po_c = (gc & 1) * NIDX

        @pl.when((g >= 1) & ((g & 1) == 1))
        def _():
            pltpu.make_async_copy(cm2.at[pl.ds(0, NIDX)],
                                  rows_v.at[pl.ds(0, NIDX)], sem0).wait()

        @pl.when((g >= 1) & ((g & 1) == 0))
        def _():
            pltpu.make_async_copy(cm2.at[pl.ds(0, NIDX)],
                                  rows_v.at[pl.ds(NIDX, NIDX)], sem1).wait()

        @pl.when(g >= 1)
        def _():
            compute(gc, po_c)

        return carry

    lax.fori_loop(0, NG + 1, step, 0)
    pltpu.sync_copy(t_v, out.at[:, pl.ds(base, PW)])


@jax.jit
def _run(cm2, x0, y0, x1, y1):
    mesh = plsc.VectorSubcoreMesh(core_axis_name="c", subcore_axis_name="s",
                                  num_cores=2, num_subcores=16)
    f = pl.kernel(
        _sc_body,
        out_type=jax.ShapeDtypeStruct((OUT_F, NP), _f32),
        mesh=mesh,
        compiler_params=pltpu.CompilerParams(use_tc_tiling_on_sc=False,
                                             needs_layout_passes=False),
        scratch_types=[
            pltpu.VMEM((PW,), _f32),
            pltpu.VMEM((PW,), _f32),
            pltpu.VMEM((PW,), _f32),
            pltpu.VMEM((PW,), _f32),
            pltpu.VMEM((2 * NIDX,), _i32),
            pltpu.VMEM((2 * NIDX, 16), _f32),
            pltpu.VMEM((1600,), _f32),
            pltpu.VMEM((OUT_F, PW), _f32),
            pltpu.SemaphoreType.DMA,
            pltpu.SemaphoreType.DMA,
        ],
    )
    return f(cm2, x0, y0, x1, y1)


def kernel(cost_maps, coords0, coords1):
    cm2 = jnp.transpose(cost_maps.reshape(NP, H, W), (1, 2, 0)).reshape(NROWS, 16)
    x0 = coords0[:, 0].reshape(-1)
    y0 = coords0[:, 1].reshape(-1)
    x1 = coords1[:, 0].reshape(-1)
    y1 = coords1[:, 1].reshape(-1)
    out = _run(cm2, x0, y0, x1, y1)
    return out.reshape(OUT_F, B, H * W).transpose(1, 0, 2).reshape(B, OUT_F, H, W)
